# baseline (speedup 1.0000x reference)
import functools

import jax
import jax.numpy as jnp
from jax import lax
from jax.experimental import pallas as pl
from jax.experimental.pallas import tpu as pltpu
from jax.experimental.pallas import tpu_sc as plsc

_C = 256
_L = 16


def _emb_lookup(flat_ids, t128, n, v, d):
    info = plsc.get_sparse_core_info()
    nw = info.num_cores * info.num_subcores
    n_per_w = n // nw
    n_ch = n_per_w // _C
    n_pairs = n_ch // 2
    pack = 128 // d
    o_rows = _C * d // 128
    mesh = plsc.VectorSubcoreMesh(core_axis_name="c", subcore_axis_name="s")

    @functools.partial(
        pl.kernel,
        mesh=mesh,
        out_type=jax.ShapeDtypeStruct((n * d // 128, 128), jnp.float32),
        scratch_types=[
            pltpu.VMEM((n_per_w,), jnp.int32),
            pltpu.VMEM((_C,), jnp.int32),
            pltpu.VMEM((_C,), jnp.int32),
            pltpu.VMEM((_C, 128), jnp.float32),
            pltpu.VMEM((_C, 128), jnp.float32),
            pltpu.VMEM((o_rows, 128), jnp.float32),
            pltpu.VMEM((o_rows, 128), jnp.float32),
            pltpu.SemaphoreType.DMA,
            pltpu.SemaphoreType.DMA,
            pltpu.SemaphoreType.DMA,
            pltpu.SemaphoreType.DMA,
        ],
        compiler_params=pltpu.CompilerParams(needs_layout_passes=False),
    )
    def k(idx_hbm, tab_hbm, out_hbm, idx_v, i4a, i4b, ga, gb, oa, ob,
          gsa, gsb, wsa, wsb):
        wid = lax.axis_index("s") * info.num_cores + lax.axis_index("c")
        base = wid * n_per_w
        obase = base * d // 128
        pltpu.sync_copy(idx_hbm.at[pl.ds(base, n_per_w)], idx_v)
        iota = lax.iota(jnp.int32, _L)

        def prep_idx4(g, i4):
            def grp(m, carry):
                iv = idx_v[pl.ds(g * _C + m * _L, _L)]
                i4[pl.ds(m * _L, _L)] = lax.shift_right_logical(iv, 2)
                return carry

            lax.fori_loop(0, _C // _L, grp, 0)

        def start_gather(i4, gbuf, sem):
            pltpu.async_copy(tab_hbm.at[i4], gbuf, sem)

        def wait_gather(i4, gbuf, sem):
            pltpu.make_async_copy(tab_hbm.at[i4], gbuf, sem).wait()

        def extract(g, gbuf, obuf):
            def grp(m, carry):
                iv = idx_v[pl.ds(g * _C + m * _L, _L)]
                col0 = lax.shift_left(jnp.bitwise_and(iv, pack - 1),
                                      jnp.int32(5))
                row = m * _L + iota
                orow = (m * (_L * d // 128)) + lax.shift_right_logical(
                    iota, jnp.int32(2))
                ocol0 = lax.shift_left(jnp.bitwise_and(iota, 3), jnp.int32(5))
                for kk in range(d):
                    vals = plsc.load_gather(gbuf, [row, col0 + kk])
                    plsc.store_scatter(obuf, [orow, ocol0 + kk], vals)
                return carry

            lax.fori_loop(0, _C // _L, grp, 0)

        def start_wb(g, obuf, sem):
            off = pl.multiple_of(obase + g * o_rows, 8)
            pltpu.async_copy(obuf, out_hbm.at[pl.ds(off, o_rows)], sem)

        def wait_wb(g, obuf, sem):
            off = pl.multiple_of(obase + g * o_rows, 8)
            pltpu.make_async_copy(obuf, out_hbm.at[pl.ds(off, o_rows)],
                                  sem).wait()

        prep_idx4(0, i4a)
        start_gather(i4a, ga, gsa)
        prep_idx4(1, i4b)
        start_gather(i4b, gb, gsb)

        def pair(p, carry):
            g0 = 2 * p
            wait_gather(i4a, ga, gsa)

            @pl.when(p > 0)
            def _():
                wait_wb(g0 - 2, oa, wsa)

            extract(g0, ga, oa)

            @pl.when(p < n_pairs - 1)
            def _():
                prep_idx4(g0 + 2, i4a)
                start_gather(i4a, ga, gsa)

            start_wb(g0, oa, wsa)

            wait_gather(i4b, gb, gsb)

            @pl.when(p > 0)
            def _():
                wait_wb(g0 - 1, ob, wsb)

            extract(g0 + 1, gb, ob)

            @pl.when(p < n_pairs - 1)
            def _():
                prep_idx4(g0 + 3, i4b)
                start_gather(i4b, gb, gsb)

            start_wb(g0 + 1, ob, wsb)
            return carry

        lax.fori_loop(0, n_pairs, pair, 0)
        wait_wb(n_ch - 2, oa, wsa)
        wait_wb(n_ch - 1, ob, wsb)

    return k(flat_ids, t128)


def kernel(token_ids, embeddings):
    b, s = token_ids.shape
    v, d = embeddings.shape
    flat = token_ids.reshape(-1).astype(jnp.int32)
    t128 = embeddings.reshape(v * d // 128, 128)
    out = _emb_lookup(flat, t128, b * s, v, d)
    return out.reshape(b, s, d)


# speedup vs baseline: 1.6274x; 1.6274x over previous
import functools

import jax
import jax.numpy as jnp
from jax import lax
from jax.experimental import pallas as pl
from jax.experimental.pallas import tpu as pltpu
from jax.experimental.pallas import tpu_sc as plsc

_C = 256
_L = 16


def _emb_lookup(flat_ids, t128, n, v, d):
    info = plsc.get_sparse_core_info()
    nw = info.num_cores * info.num_subcores
    n_per_w = n // nw
    n_ch = n_per_w // _C
    n_pairs = n_ch // 2
    pack = 128 // d
    o_rows = _C * d // 128
    mesh = plsc.VectorSubcoreMesh(core_axis_name="c", subcore_axis_name="s")

    @functools.partial(
        pl.kernel,
        mesh=mesh,
        out_type=jax.ShapeDtypeStruct((n * d // 128, 128), jnp.float32),
        scratch_types=[
            pltpu.VMEM((n_per_w,), jnp.int32),
            pltpu.VMEM((_C,), jnp.int32),
            pltpu.VMEM((_C,), jnp.int32),
            pltpu.VMEM((_C, 128), jnp.float32),
            pltpu.VMEM((_C, 128), jnp.float32),
            pltpu.VMEM((o_rows, 128), jnp.float32),
            pltpu.VMEM((o_rows, 128), jnp.float32),
            pltpu.SemaphoreType.DMA,
            pltpu.SemaphoreType.DMA,
            pltpu.SemaphoreType.DMA,
            pltpu.SemaphoreType.DMA,
        ],
        compiler_params=pltpu.CompilerParams(needs_layout_passes=False),
    )
    def k(idx_hbm, tab_hbm, out_hbm, idx_v, i4a, i4b, ga, gb, oa, ob,
          gsa, gsb, wsa, wsb):
        wid = lax.axis_index("s") * info.num_cores + lax.axis_index("c")
        base = wid * n_per_w
        obase = base * d // 128
        pltpu.sync_copy(idx_hbm.at[pl.ds(base, n_per_w)], idx_v)
        iota = lax.iota(jnp.int32, _L)

        def prep_idx4(g, i4):
            def grp(m, carry):
                iv = idx_v[pl.ds(g * _C + m * _L, _L)]
                i4[pl.ds(m * _L, _L)] = lax.shift_right_logical(iv, 2)
                return carry

            lax.fori_loop(0, _C // _L, grp, 0)

        def start_gather(i4, gbuf, sem):
            pltpu.async_copy(tab_hbm.at[i4], gbuf, sem)

        def wait_gather(i4, gbuf, sem):
            pltpu.make_async_copy(tab_hbm.at[i4], gbuf, sem).wait()

        def extract(g, gbuf, obuf):
            def grp(m, carry):
                iv = idx_v[pl.ds(g * _C + m * _L, _L)]
                for t in range(_L):
                    col0 = lax.shift_left(
                        jnp.bitwise_and(iv[t], pack - 1), jnp.int32(5))
                    j = m * _L + t
                    orow = m * (_L * d // 128) + (t * d) // 128
                    ocol = (t * d) % 128
                    for h in range(d // _L):
                        vals = gbuf[j, pl.ds(col0 + h * _L, _L)]
                        obuf[orow, pl.ds(ocol + h * _L, _L)] = vals
                return carry

            lax.fori_loop(0, _C // _L, grp, 0)

        def start_wb(g, obuf, sem):
            off = pl.multiple_of(obase + g * o_rows, 8)
            pltpu.async_copy(obuf, out_hbm.at[pl.ds(off, o_rows)], sem)

        def wait_wb(g, obuf, sem):
            off = pl.multiple_of(obase + g * o_rows, 8)
            pltpu.make_async_copy(obuf, out_hbm.at[pl.ds(off, o_rows)],
                                  sem).wait()

        prep_idx4(0, i4a)
        start_gather(i4a, ga, gsa)
        prep_idx4(1, i4b)
        start_gather(i4b, gb, gsb)

        def pair(p, carry):
            g0 = 2 * p
            wait_gather(i4a, ga, gsa)

            @pl.when(p > 0)
            def _():
                wait_wb(g0 - 2, oa, wsa)

            extract(g0, ga, oa)

            @pl.when(p < n_pairs - 1)
            def _():
                prep_idx4(g0 + 2, i4a)
                start_gather(i4a, ga, gsa)

            start_wb(g0, oa, wsa)

            wait_gather(i4b, gb, gsb)

            @pl.when(p > 0)
            def _():
                wait_wb(g0 - 1, ob, wsb)

            extract(g0 + 1, gb, ob)

            @pl.when(p < n_pairs - 1)
            def _():
                prep_idx4(g0 + 3, i4b)
                start_gather(i4b, gb, gsb)

            start_wb(g0 + 1, ob, wsb)
            return carry

        lax.fori_loop(0, n_pairs, pair, 0)
        wait_wb(n_ch - 2, oa, wsa)
        wait_wb(n_ch - 1, ob, wsb)

    return k(flat_ids, t128)


def kernel(token_ids, embeddings):
    b, s = token_ids.shape
    v, d = embeddings.shape
    flat = token_ids.reshape(-1).astype(jnp.int32)
    t128 = embeddings.reshape(v * d // 128, 128)
    out = _emb_lookup(flat, t128, b * s, v, d)
    return out.reshape(b, s, d)


# speedup vs baseline: 1.7477x; 1.0739x over previous
import functools

import jax
import jax.numpy as jnp
from jax import lax
from jax.experimental import pallas as pl
from jax.experimental.pallas import tpu as pltpu
from jax.experimental.pallas import tpu_sc as plsc

_SENT_PER_CHUNK = 4
_L = 16


def _emb_lookup(flat_ids, t128, b, s, v, d):
    n = b * s
    info = plsc.get_sparse_core_info()
    nw = info.num_cores * info.num_subcores
    n_per_w = n // nw
    c = _SENT_PER_CHUNK * s
    n_ch = n_per_w // c
    n_pairs = n_ch // 2
    pack = 128 // d
    n_grp = (c + _L - 1) // _L
    c_pad = n_grp * _L
    mesh = plsc.VectorSubcoreMesh(core_axis_name="c", subcore_axis_name="s")

    @functools.partial(
        pl.kernel,
        mesh=mesh,
        out_type=jax.ShapeDtypeStruct((b, s, d), jnp.float32),
        scratch_types=[
            pltpu.VMEM((n_per_w,), jnp.int32),
            pltpu.VMEM((c,), jnp.int32),
            pltpu.VMEM((c,), jnp.int32),
            pltpu.VMEM((c, 128), jnp.float32),
            pltpu.VMEM((c, 128), jnp.float32),
            pltpu.VMEM((c, d), jnp.float32),
            pltpu.VMEM((c, d), jnp.float32),
            pltpu.SemaphoreType.DMA,
            pltpu.SemaphoreType.DMA,
            pltpu.SemaphoreType.DMA,
            pltpu.SemaphoreType.DMA,
        ],
        compiler_params=pltpu.CompilerParams(needs_layout_passes=False),
    )
    def k(idx_hbm, tab_hbm, out_hbm, idx_v, i4a, i4b, ga, gb, oa, ob,
          gsa, gsb, wsa, wsb):
        wid = lax.axis_index("s") * info.num_cores + lax.axis_index("c")
        base = wid * n_per_w
        sbase = base // s
        pltpu.sync_copy(idx_hbm.at[pl.ds(base, n_per_w)], idx_v)
        iota = lax.iota(jnp.int32, _L)

        def prep_idx4(g, i4):
            def grp(m, carry):
                iv = idx_v[pl.ds(g * c + m * _L, _L)]
                i4[pl.ds(m * _L, _L)] = lax.shift_right_logical(iv, 2)
                return carry

            lax.fori_loop(0, n_grp - 1, grp, 0)
            iv = idx_v[pl.ds(g * c + (c - _L), _L)]
            i4[pl.ds(c - _L, _L)] = lax.shift_right_logical(iv, 2)

        def start_gather(i4, gbuf, sem):
            pltpu.async_copy(tab_hbm.at[i4], gbuf, sem)

        def wait_gather(i4, gbuf, sem):
            pltpu.make_async_copy(tab_hbm.at[i4], gbuf, sem).wait()

        def extract(g, gbuf, obuf):
            def do_token_at(iv, tok0, t):
                tok = tok0 + t
                col0 = lax.shift_left(
                    jnp.bitwise_and(iv[t], pack - 1), jnp.int32(5))
                for h in range(d // _L):
                    obuf[tok, pl.ds(h * _L, _L)] = (
                        gbuf[tok, pl.ds(col0 + h * _L, _L)])

            def grp(m, carry):
                iv = idx_v[pl.ds(g * c + m * _L, _L)]
                for t in range(_L):
                    do_token_at(iv, m * _L, t)
                return carry

            lax.fori_loop(0, n_grp - 1, grp, 0)
            iv = idx_v[pl.ds(g * c + (c - _L), _L)]
            for t in range(_L - (c_pad - c), _L):
                do_token_at(iv, c - _L, t)

        def start_wb(g, obuf, sem):
            off = pl.multiple_of(sbase + g * _SENT_PER_CHUNK, _SENT_PER_CHUNK)
            pltpu.async_copy(obuf.reshape(_SENT_PER_CHUNK, s, d),
                             out_hbm.at[pl.ds(off, _SENT_PER_CHUNK)], sem)

        def wait_wb(g, obuf, sem):
            off = pl.multiple_of(sbase + g * _SENT_PER_CHUNK, _SENT_PER_CHUNK)
            pltpu.make_async_copy(
                obuf.reshape(_SENT_PER_CHUNK, s, d),
                out_hbm.at[pl.ds(off, _SENT_PER_CHUNK)], sem).wait()

        prep_idx4(0, i4a)
        start_gather(i4a, ga, gsa)
        prep_idx4(1, i4b)
        start_gather(i4b, gb, gsb)

        def pair(p, carry):
            g0 = 2 * p
            wait_gather(i4a, ga, gsa)

            @pl.when(p > 0)
            def _():
                wait_wb(g0 - 2, oa, wsa)

            extract(g0, ga, oa)

            @pl.when(p < n_pairs - 1)
            def _():
                prep_idx4(g0 + 2, i4a)
                start_gather(i4a, ga, gsa)

            start_wb(g0, oa, wsa)

            wait_gather(i4b, gb, gsb)

            @pl.when(p > 0)
            def _():
                wait_wb(g0 - 1, ob, wsb)

            extract(g0 + 1, gb, ob)

            @pl.when(p < n_pairs - 1)
            def _():
                prep_idx4(g0 + 3, i4b)
                start_gather(i4b, gb, gsb)

            start_wb(g0 + 1, ob, wsb)
            return carry

        lax.fori_loop(0, n_pairs, pair, 0)
        wait_wb(n_ch - 2, oa, wsa)
        wait_wb(n_ch - 1, ob, wsb)

    return k(flat_ids, t128)


def kernel(token_ids, embeddings):
    b, s = token_ids.shape
    v, d = embeddings.shape
    flat = token_ids.reshape(-1).astype(jnp.int32)
    t128 = embeddings.reshape(v * d // 128, 128)
    return _emb_lookup(flat, t128, b, s, v, d)
